# baseline (speedup 1.0000x reference)
import functools

import jax
import jax.numpy as jnp
from jax import lax
from jax.experimental import pallas as pl
from jax.experimental.pallas import tpu as pltpu
from jax.experimental.pallas import tpu_sc as plsc

_N = 4096
_M = 1024
_B = 4
_G = 8
_C = 32
_MG = _M * _G
_PB = _N // _B
_TAB = _PB + 16
_SENT = _PB + 3
_MAXD2 = 2.4 * 2.4
_R2 = (2.4 * 2.0) * (2.4 * 2.0)
_BIG = 1e10
_INF = 1e30

_NW = 32
_CPW = _MG // _NW
_NG = _CPW // 16
_FTW = _C * _TAB


def _rsqrt(x):
    xc = jnp.maximum(x, jnp.float32(4.0 ** -11))
    y = jnp.full((16,), 0.35355339, jnp.float32)
    for k in range(1, 13):
        y = jnp.where(xc < 4.0 ** (2 - k),
                      jnp.float32(1.41421356 * 2.0 ** (k - 2)), y)
    y = jnp.where(xc > 1e9, jnp.float32(1e-5), y)
    xh = xc * 0.5
    for _ in range(5):
        y = y * (1.5 - xh * y * y)
    return y


def _body(cp_h, sp_h, ft_h, ws_h, out_h,
          gx_v, gy_v, gz_v, bgx_v, bgy_v, bgz_v, bnx_v, bny_v, bnz_v, sqn_v,
          sx_v, sy_v, sz_v, bsx_v, bsy_v, bsz_v, sqs_v, cl_v, ft_v, ws_v,
          out_v, dma_sem):
    wid = lax.axis_index("s") * 2 + lax.axis_index("c")
    b = wid // 8
    cbase = wid * _CPW

    copies = []
    cdsts = (gx_v, gy_v, gz_v, bgx_v, bgy_v, bgz_v, bnx_v, bny_v, bnz_v, sqn_v)
    for k, dst in enumerate(cdsts):
        copies.append(pltpu.async_copy(
            cp_h.at[pl.ds(k * _MG + cbase, _CPW)], dst, dma_sem))
    sdsts = (sx_v, sy_v, sz_v, bsx_v, bsy_v, bsz_v)
    for k, dst in enumerate(sdsts):
        copies.append(pltpu.async_copy(
            sp_h.at[pl.ds((k * _B + b) * _TAB, _TAB)], dst, dma_sem))
    copies.append(pltpu.async_copy(ft_h.at[pl.ds(b * _FTW, _FTW)], ft_v, dma_sem))
    copies.append(pltpu.async_copy(ws_h, ws_v, dma_sem))
    for cpy in copies:
        cpy.wait()

    def sqs_step(t, _):
        xv = sx_v[pl.ds(t * 16, 16)]
        yv = sy_v[pl.ds(t * 16, 16)]
        zv = sz_v[pl.ds(t * 16, 16)]
        sqs_v[pl.ds(t * 16, 16)] = (xv * xv + yv * yv) + zv * zv
        return 0

    lax.fori_loop(0, _TAB // 16, sqs_step, 0)

    is_b0 = (b == 0)
    m_init = jnp.where(is_b0, _INF, _BIG)

    def group(g, _):
        gxv = gx_v[pl.ds(g * 16, 16)]
        gyv = gy_v[pl.ds(g * 16, 16)]
        gzv = gz_v[pl.ds(g * 16, 16)]
        bgxv = bgx_v[pl.ds(g * 16, 16)]
        bgyv = bgy_v[pl.ds(g * 16, 16)]
        bgzv = bgz_v[pl.ds(g * 16, 16)]
        bnxv = bnx_v[pl.ds(g * 16, 16)]
        bnyv = bny_v[pl.ds(g * 16, 16)]
        bnzv = bnz_v[pl.ds(g * 16, 16)]
        sqnv = sqn_v[pl.ds(g * 16, 16)]
        sqgv = (gxv * gxv + gyv * gyv) + gzv * gzv

        bnxA = jnp.broadcast_to(bnxv[0], (16,))
        bnyA = jnp.broadcast_to(bnyv[0], (16,))
        bnzA = jnp.broadcast_to(bnzv[0], (16,))
        sqnA = jnp.broadcast_to(sqnv[0], (16,))
        bnxB = jnp.broadcast_to(bnxv[8], (16,))
        bnyB = jnp.broadcast_to(bnyv[8], (16,))
        bnzB = jnp.broadcast_to(bnzv[8], (16,))
        sqnB = jnp.broadcast_to(sqnv[8], (16,))
        iota = lax.iota(jnp.int32, 16)

        def p1(t, off):
            for u in range(2):
                base = (t * 2 + u) * 16
                svx = bsx_v[pl.ds(base, 16)]
                svy = bsy_v[pl.ds(base, 16)]
                svz = bsz_v[pl.ds(base, 16)]
                svq = sqs_v[pl.ds(base, 16)]
                dotA = (bnxA * svx + bnyA * svy) + bnzA * svz
                dA = (sqnA + svq) - (dotA + dotA)
                dotB = (bnxB * svx + bnyB * svy) + bnzB * svz
                dB = (sqnB + svq) - (dotB + dotB)
                mA = dA <= _R2
                mB = dB <= _R2
                mk = mA | mB
                idxv = (jnp.full((16,), base, jnp.int32) + iota
                        + jnp.where(mA, 65536, 0) + jnp.where(mB, 131072, 0))
                plsc.store_compressed(cl_v.at[pl.ds(off, 16)], idxv, mask=mk)
                pc = plsc.all_reduce_population_count(mk)
                off = off + pc[0]
            return off

        n_cand = lax.fori_loop(0, _PB // 32, p1, jnp.int32(0))
        cl_v[pl.ds(n_cand, 16)] = jnp.full((16,), _SENT, jnp.int32)

        def eval_cand(bsx, bsy, bsz, sqs):
            dot = (bgxv * bsx + bgyv * bsy) + bgzv * bsz
            d2 = (sqgv + sqs) - (dot + dot)
            dotc = (bnxv * bsx + bnyv * bsy) + bnzv * bsz
            d2c = (sqnv + sqs) - (dotc + dotc)
            valid = (d2 <= _MAXD2) & (d2c <= _R2)
            return valid, d2

        def insert(carry, key, jv):
            m1, m2, m3, i1, i2, i3 = carry
            c1 = key < m1
            c2 = key < m2
            c3 = key < m3
            m3n = jnp.where(c2, m2, jnp.where(c3, key, m3))
            i3n = jnp.where(c2, i2, jnp.where(c3, jv, i3))
            m2n = jnp.where(c1, m1, jnp.where(c2, key, m2))
            i2n = jnp.where(c1, i1, jnp.where(c2, jv, i2))
            m1 = jnp.where(c1, key, m1)
            i1 = jnp.where(c1, jv, i1)
            return m1, m2n, m3n, i1, i2n, i3n

        m0 = jnp.full((16,), m_init, jnp.float32)
        carry = (m0, m0, m0,
                 jnp.full((16,), _PB, jnp.int32),
                 jnp.full((16,), _PB + 1, jnp.int32),
                 jnp.full((16,), _PB + 2, jnp.int32))

        svx0 = bsx_v[pl.ds(0, 16)]
        svy0 = bsy_v[pl.ds(0, 16)]
        svz0 = bsz_v[pl.ds(0, 16)]
        svq0 = sqs_v[pl.ds(0, 16)]
        for u in range(6):
            valid, _ = eval_cand(jnp.broadcast_to(svx0[u], (16,)),
                                 jnp.broadcast_to(svy0[u], (16,)),
                                 jnp.broadcast_to(svz0[u], (16,)),
                                 jnp.broadcast_to(svq0[u], (16,)))
            key = jnp.where(valid, _INF, _BIG)
            carry = insert(carry, key, jnp.full((16,), u, jnp.int32))

        halfsel = jnp.where(iota < 8, jnp.int32(65536), jnp.int32(131072))

        def p2(t, carry):
            civ = cl_v[pl.ds(t * 16, 16)]
            jv16 = civ & 0xFFFF
            cxv = plsc.load_gather(bsx_v, [jv16])
            cyv = plsc.load_gather(bsy_v, [jv16])
            czv = plsc.load_gather(bsz_v, [jv16])
            cqv = plsc.load_gather(sqs_v, [jv16])
            for u in range(16):
                bsx = jnp.broadcast_to(cxv[u], (16,))
                bsy = jnp.broadcast_to(cyv[u], (16,))
                bsz = jnp.broadcast_to(czv[u], (16,))
                sqs = jnp.broadcast_to(cqv[u], (16,))
                dot = (bgxv * bsx + bgyv * bsy) + bgzv * bsz
                d2 = (sqgv + sqs) - (dot + dot)
                pok = (jnp.broadcast_to(civ[u], (16,)) & halfsel) != 0
                valid = (d2 <= _MAXD2) & pok
                key = jnp.where(valid, d2, _BIG)
                carry = insert(carry, key, jnp.broadcast_to(jv16[u], (16,)))
            return carry

        nblk = (n_cand + 15) // 16
        m1, m2, m3, i1, i2, i3 = lax.fori_loop(0, nblk, p2, carry)

        empty = m1 >= (_BIG * 0.5)
        zero = jnp.zeros((16,), jnp.float32)
        ws = []
        for mk in (m1, m2, m3):
            mk0 = jnp.maximum(mk, 0.0)
            dist = mk0 * _rsqrt(mk0)
            ws.append(1.0 / (dist + 1e-8))
        norm = jnp.maximum(ws[0] + ws[1] + ws[2], 1e-8)
        w1 = jnp.where(empty, zero, ws[0] / norm)
        w2 = jnp.where(empty, zero, ws[1] / norm)
        w3 = jnp.where(empty, zero, ws[2] / norm)

        loc = []
        for ik in (i1, i2, i3):
            nx = plsc.load_gather(sx_v, [ik])
            ny = plsc.load_gather(sy_v, [ik])
            nz = plsc.load_gather(sz_v, [ik])
            loc.append(jnp.where(empty, zero, gxv - nx))
            loc.append(jnp.where(empty, zero, gyv - ny))
            loc.append(jnp.where(empty, zero, gzv - nz))

        def outcol(t, _):
            for u in range(2):
                c = t * 2 + u
                off = jnp.full((16,), c * _TAB, jnp.int32)
                p1 = plsc.load_gather(ft_v, [i1 + off])
                p2 = plsc.load_gather(ft_v, [i2 + off])
                p3 = plsc.load_gather(ft_v, [i3 + off])
                acc = (p1 * w1 + p2 * w2) + p3 * w3
                wv = ws_v[pl.ds(c * 16, 16)]
                a0 = (loc[0] * jnp.broadcast_to(wv[0], (16,))
                      + loc[1] * jnp.broadcast_to(wv[1], (16,))
                      + loc[2] * jnp.broadcast_to(wv[2], (16,)))
                a1 = (loc[3] * jnp.broadcast_to(wv[3], (16,))
                      + loc[4] * jnp.broadcast_to(wv[4], (16,))
                      + loc[5] * jnp.broadcast_to(wv[5], (16,)))
                a2 = (loc[6] * jnp.broadcast_to(wv[6], (16,))
                      + loc[7] * jnp.broadcast_to(wv[7], (16,))
                      + loc[8] * jnp.broadcast_to(wv[8], (16,)))
                acc = ((acc + a0) + (a1 + a2)) + jnp.broadcast_to(wv[9], (16,))
                out_v[pl.ds(c * _CPW + g * 16, 16)] = jnp.maximum(acc, 0.0)
            return 0

        lax.fori_loop(0, _C // 2, outcol, 0)
        return 0

    lax.fori_loop(0, _NG, group, 0)
    pltpu.sync_copy(out_v, out_h.at[pl.ds(wid * (_C * _CPW), _C * _CPW)])


def _proj_body(sf_ref, w_ref, out_ref):
    out_ref[...] = jax.lax.dot_general(
        sf_ref[...], w_ref[...], (((1,), (0,)), ((), ())),
        precision=jax.lax.Precision.HIGHEST,
        preferred_element_type=jnp.float32)


def _proj(sf, wf):
    return pl.pallas_call(
        _proj_body,
        out_shape=jax.ShapeDtypeStruct((_N, _C), jnp.float32),
    )(sf, wf)


@jax.jit
def _run(cpack, spack, fa, wsf):
    mesh = plsc.VectorSubcoreMesh(core_axis_name="c", subcore_axis_name="s")
    k = functools.partial(
        pl.kernel, _body, mesh=mesh,
        compiler_params=pltpu.CompilerParams(needs_layout_passes=False),
        out_type=jax.ShapeDtypeStruct((_NW * _C * _CPW,), jnp.float32),
        scratch_types=[pltpu.VMEM((_CPW,), jnp.float32)] * 10
        + [pltpu.VMEM((_TAB,), jnp.float32)] * 6
        + [
            pltpu.VMEM((_TAB,), jnp.float32),
            pltpu.VMEM((_TAB,), jnp.int32),
            pltpu.VMEM((_FTW,), jnp.float32),
            pltpu.VMEM((_C * 16,), jnp.float32),
            pltpu.VMEM((_C * _CPW,), jnp.float32),
            pltpu.SemaphoreType.DMA,
        ],
    )()
    return k(cpack, spack, fa, wsf)


def kernel(support_xyz, support_features, xyz_batch_cnt, new_xyz,
           new_xyz_grid_centers, new_xyz_batch_cnt, W, gamma, beta):
    def bf(x):
        return lax.reduce_precision(x, exponent_bits=8, mantissa_bits=7)

    gcf = new_xyz_grid_centers.reshape(_MG, 3)
    bgc = bf(gcf)

    par = jnp.repeat(new_xyz, _G, axis=0)
    sqn = (par[:, 0] * par[:, 0] + par[:, 1] * par[:, 1]) + par[:, 2] * par[:, 2]
    bpar = bf(par)

    cpack = jnp.concatenate(
        [gcf.T, bgc.T, bpar.T, sqn[None]], axis=0).reshape(-1)

    sb = support_xyz.reshape(_B, _PB, 3)
    extra = jnp.broadcast_to(support_xyz[0:3][None], (_B, 3, 3))
    padx = jnp.full((_B, _TAB - _PB - 3, 3), 1e6, jnp.float32)
    sa = jnp.concatenate([sb, extra, padx], axis=1)
    bsa = bf(sa)
    spack = jnp.concatenate(
        [sa.transpose(2, 0, 1), bsa.transpose(2, 0, 1)], axis=0).reshape(-1)

    scale = gamma * (1.0 / jnp.sqrt(jnp.float32(1.0 + 1e-5)))
    proj = _proj(support_features, W[:_C] * scale[None, :])

    fb = proj.reshape(_B, _PB, _C).transpose(0, 2, 1)
    fextra = jnp.broadcast_to(proj[0:3].T[None], (_B, _C, 3))
    fpad = jnp.zeros((_B, _C, _TAB - _PB - 3), jnp.float32)
    fa = jnp.concatenate([fb, fextra, fpad], axis=2).reshape(-1)

    wloc = jnp.concatenate(
        [(W[_C:] * scale[None, :]).T, beta[:, None],
         jnp.zeros((_C, 6), jnp.float32)], axis=1).reshape(-1)

    out = _run(cpack, spack, fa, wloc)
    return out.reshape(_NW, _C, _CPW).transpose(0, 2, 1).reshape(_MG, _C)


# speedup vs baseline: 1.0351x; 1.0351x over previous
import functools

import jax
import jax.numpy as jnp
from jax import lax
from jax.experimental import pallas as pl
from jax.experimental.pallas import tpu as pltpu
from jax.experimental.pallas import tpu_sc as plsc

_N = 4096
_M = 1024
_B = 4
_G = 8
_C = 32
_MG = _M * _G
_PB = _N // _B
_TAB = _PB + 16
_SENT = _PB + 3
_MAXD2 = 2.4 * 2.4
_R2 = (2.4 * 2.0) * (2.4 * 2.0)
_BIG = 1e10
_INF = 1e30

_NW = 32
_CPW = _MG // _NW
_NG = _CPW // 16
_FTW = _C * _TAB


def _rsqrt(x):
    xc = jnp.maximum(x, jnp.float32(4.0 ** -11))
    y = jnp.full((16,), 0.35355339, jnp.float32)
    for k in range(1, 13):
        y = jnp.where(xc < 4.0 ** (2 - k),
                      jnp.float32(1.41421356 * 2.0 ** (k - 2)), y)
    y = jnp.where(xc > 1e9, jnp.float32(1e-5), y)
    xh = xc * 0.5
    for _ in range(5):
        y = y * (1.5 - xh * y * y)
    return y


def _body(cp_h, sp_h, ft_h, ws_h, out_h,
          gx_v, gy_v, gz_v, bgx_v, bgy_v, bgz_v, bnx_v, bny_v, bnz_v, sqn_v,
          sx_v, sy_v, sz_v, bsx_v, bsy_v, bsz_v, sqs_v, cl_v, ft_v, ws_v,
          out_v, dma_sem):
    wid = lax.axis_index("s") * 2 + lax.axis_index("c")
    b = wid // 8
    cbase = wid * _CPW

    copies = []
    cdsts = (gx_v, gy_v, gz_v, bgx_v, bgy_v, bgz_v, bnx_v, bny_v, bnz_v, sqn_v)
    for k, dst in enumerate(cdsts):
        copies.append(pltpu.async_copy(
            cp_h.at[pl.ds(k * _MG + cbase, _CPW)], dst, dma_sem))
    sdsts = (sx_v, sy_v, sz_v, bsx_v, bsy_v, bsz_v)
    for k, dst in enumerate(sdsts):
        copies.append(pltpu.async_copy(
            sp_h.at[pl.ds((k * _B + b) * _TAB, _TAB)], dst, dma_sem))
    copies.append(pltpu.async_copy(ft_h.at[pl.ds(b * _FTW, _FTW)], ft_v, dma_sem))
    copies.append(pltpu.async_copy(ws_h, ws_v, dma_sem))
    for cpy in copies:
        cpy.wait()

    def sqs_step(t, _):
        xv = sx_v[pl.ds(t * 16, 16)]
        yv = sy_v[pl.ds(t * 16, 16)]
        zv = sz_v[pl.ds(t * 16, 16)]
        sqs_v[pl.ds(t * 16, 16)] = (xv * xv + yv * yv) + zv * zv
        return 0

    lax.fori_loop(0, _TAB // 16, sqs_step, 0)

    is_b0 = (b == 0)
    m_init = jnp.where(is_b0, _INF, _BIG)

    def group(g, _):
        gxv = gx_v[pl.ds(g * 16, 16)]
        gyv = gy_v[pl.ds(g * 16, 16)]
        gzv = gz_v[pl.ds(g * 16, 16)]
        bgxv = bgx_v[pl.ds(g * 16, 16)]
        bgyv = bgy_v[pl.ds(g * 16, 16)]
        bgzv = bgz_v[pl.ds(g * 16, 16)]
        bnxv = bnx_v[pl.ds(g * 16, 16)]
        bnyv = bny_v[pl.ds(g * 16, 16)]
        bnzv = bnz_v[pl.ds(g * 16, 16)]
        sqnv = sqn_v[pl.ds(g * 16, 16)]
        sqgv = (gxv * gxv + gyv * gyv) + gzv * gzv

        bnxA = jnp.broadcast_to(bnxv[0], (16,))
        bnyA = jnp.broadcast_to(bnyv[0], (16,))
        bnzA = jnp.broadcast_to(bnzv[0], (16,))
        sqnA = jnp.broadcast_to(sqnv[0], (16,))
        bnxB = jnp.broadcast_to(bnxv[8], (16,))
        bnyB = jnp.broadcast_to(bnyv[8], (16,))
        bnzB = jnp.broadcast_to(bnzv[8], (16,))
        sqnB = jnp.broadcast_to(sqnv[8], (16,))
        iota = lax.iota(jnp.int32, 16)

        def p1(t, off):
            for u in range(2):
                base = (t * 2 + u) * 16
                svx = bsx_v[pl.ds(base, 16)]
                svy = bsy_v[pl.ds(base, 16)]
                svz = bsz_v[pl.ds(base, 16)]
                svq = sqs_v[pl.ds(base, 16)]
                dotA = (bnxA * svx + bnyA * svy) + bnzA * svz
                dA = (sqnA + svq) - (dotA + dotA)
                dotB = (bnxB * svx + bnyB * svy) + bnzB * svz
                dB = (sqnB + svq) - (dotB + dotB)
                mA = dA <= _R2
                mB = dB <= _R2
                mk = mA | mB
                idxv = (jnp.full((16,), base, jnp.int32) + iota
                        + jnp.where(mA, 65536, 0) + jnp.where(mB, 131072, 0))
                plsc.store_compressed(cl_v.at[pl.ds(off, 16)], idxv, mask=mk)
                pc = plsc.all_reduce_population_count(mk)
                off = off + pc[0]
            return off

        n_cand = lax.fori_loop(0, _PB // 32, p1, jnp.int32(0))
        cl_v[pl.ds(n_cand, 16)] = jnp.full((16,), _SENT, jnp.int32)

        def eval_cand(bsx, bsy, bsz, sqs):
            dot = (bgxv * bsx + bgyv * bsy) + bgzv * bsz
            d2 = (sqgv + sqs) - (dot + dot)
            dotc = (bnxv * bsx + bnyv * bsy) + bnzv * bsz
            d2c = (sqnv + sqs) - (dotc + dotc)
            valid = (d2 <= _MAXD2) & (d2c <= _R2)
            return valid, d2

        def insert(carry, key, jv):
            m1, m2, m3, i1, i2, i3 = carry
            c1 = key < m1
            c2 = key < m2
            c3 = key < m3
            m3n = jnp.where(c2, m2, jnp.where(c3, key, m3))
            i3n = jnp.where(c2, i2, jnp.where(c3, jv, i3))
            m2n = jnp.where(c1, m1, jnp.where(c2, key, m2))
            i2n = jnp.where(c1, i1, jnp.where(c2, jv, i2))
            m1 = jnp.where(c1, key, m1)
            i1 = jnp.where(c1, jv, i1)
            return m1, m2n, m3n, i1, i2n, i3n

        m0 = jnp.full((16,), m_init, jnp.float32)
        carry = (m0, m0, m0,
                 jnp.full((16,), _PB, jnp.int32),
                 jnp.full((16,), _PB + 1, jnp.int32),
                 jnp.full((16,), _PB + 2, jnp.int32))

        svx0 = bsx_v[pl.ds(0, 16)]
        svy0 = bsy_v[pl.ds(0, 16)]
        svz0 = bsz_v[pl.ds(0, 16)]
        svq0 = sqs_v[pl.ds(0, 16)]
        for u in range(6):
            valid, _ = eval_cand(jnp.broadcast_to(svx0[u], (16,)),
                                 jnp.broadcast_to(svy0[u], (16,)),
                                 jnp.broadcast_to(svz0[u], (16,)),
                                 jnp.broadcast_to(svq0[u], (16,)))
            key = jnp.where(valid, _INF, _BIG)
            carry = insert(carry, key, jnp.full((16,), u, jnp.int32))

        halfsel = jnp.where(iota < 8, jnp.int32(65536), jnp.int32(131072))

        def p2(t, carry):
            civ = cl_v[pl.ds(t * 16, 16)]
            jv16 = civ & 0xFFFF
            cxv = plsc.load_gather(bsx_v, [jv16])
            cyv = plsc.load_gather(bsy_v, [jv16])
            czv = plsc.load_gather(bsz_v, [jv16])
            cqv = plsc.load_gather(sqs_v, [jv16])
            for u in range(16):
                bsx = jnp.broadcast_to(cxv[u], (16,))
                bsy = jnp.broadcast_to(cyv[u], (16,))
                bsz = jnp.broadcast_to(czv[u], (16,))
                sqs = jnp.broadcast_to(cqv[u], (16,))
                dot = (bgxv * bsx + bgyv * bsy) + bgzv * bsz
                d2 = (sqgv + sqs) - (dot + dot)
                pok = (jnp.broadcast_to(civ[u], (16,)) & halfsel) != 0
                valid = (d2 <= _MAXD2) & pok
                key = jnp.where(valid, d2, _BIG)
                carry = insert(carry, key, jnp.broadcast_to(jv16[u], (16,)))
            return carry

        nblk = (n_cand + 15) // 16
        m1, m2, m3, i1, i2, i3 = lax.fori_loop(0, nblk, p2, carry)

        empty = m1 >= (_BIG * 0.5)
        zero = jnp.zeros((16,), jnp.float32)
        ws = []
        for mk in (m1, m2, m3):
            mk0 = jnp.maximum(mk, 0.0)
            dist = mk0 * _rsqrt(mk0)
            ws.append(1.0 / (dist + 1e-8))
        norm = jnp.maximum(ws[0] + ws[1] + ws[2], 1e-8)
        w1 = jnp.where(empty, zero, ws[0] / norm)
        w2 = jnp.where(empty, zero, ws[1] / norm)
        w3 = jnp.where(empty, zero, ws[2] / norm)

        loc = []
        for ik in (i1, i2, i3):
            nx = plsc.load_gather(sx_v, [ik])
            ny = plsc.load_gather(sy_v, [ik])
            nz = plsc.load_gather(sz_v, [ik])
            loc.append(jnp.where(empty, zero, gxv - nx))
            loc.append(jnp.where(empty, zero, gyv - ny))
            loc.append(jnp.where(empty, zero, gzv - nz))

        def outcol(c, _):
            off = jnp.full((16,), c * _TAB, jnp.int32)
            p1 = plsc.load_gather(ft_v, [i1 + off])
            p2 = plsc.load_gather(ft_v, [i2 + off])
            p3 = plsc.load_gather(ft_v, [i3 + off])
            acc = (p1 * w1 + p2 * w2) + p3 * w3
            wv = ws_v[pl.ds(c * 16, 16)]
            a0 = (loc[0] * jnp.broadcast_to(wv[0], (16,))
                  + loc[1] * jnp.broadcast_to(wv[1], (16,))
                  + loc[2] * jnp.broadcast_to(wv[2], (16,)))
            a1 = (loc[3] * jnp.broadcast_to(wv[3], (16,))
                  + loc[4] * jnp.broadcast_to(wv[4], (16,))
                  + loc[5] * jnp.broadcast_to(wv[5], (16,)))
            a2 = (loc[6] * jnp.broadcast_to(wv[6], (16,))
                  + loc[7] * jnp.broadcast_to(wv[7], (16,))
                  + loc[8] * jnp.broadcast_to(wv[8], (16,)))
            acc = ((acc + a0) + (a1 + a2)) + jnp.broadcast_to(wv[9], (16,))
            out_v[pl.ds(c * _CPW + g * 16, 16)] = jnp.maximum(acc, 0.0)
            return 0

        lax.fori_loop(0, _C, outcol, 0)
        return 0

    lax.fori_loop(0, _NG, group, 0)
    pltpu.sync_copy(out_v, out_h.at[pl.ds(wid * (_C * _CPW), _C * _CPW)])


def _proj_body(sf_ref, w_ref, out_ref):
    out_ref[...] = jax.lax.dot_general(
        sf_ref[...], w_ref[...], (((1,), (0,)), ((), ())),
        precision=jax.lax.Precision.HIGHEST,
        preferred_element_type=jnp.float32)


def _proj(sf, wf):
    return pl.pallas_call(
        _proj_body,
        out_shape=jax.ShapeDtypeStruct((_N, _C), jnp.float32),
    )(sf, wf)


@jax.jit
def _run(cpack, spack, fa, wsf):
    mesh = plsc.VectorSubcoreMesh(core_axis_name="c", subcore_axis_name="s")
    k = functools.partial(
        pl.kernel, _body, mesh=mesh,
        compiler_params=pltpu.CompilerParams(needs_layout_passes=False),
        out_type=jax.ShapeDtypeStruct((_NW * _C * _CPW,), jnp.float32),
        scratch_types=[pltpu.VMEM((_CPW,), jnp.float32)] * 10
        + [pltpu.VMEM((_TAB,), jnp.float32)] * 6
        + [
            pltpu.VMEM((_TAB,), jnp.float32),
            pltpu.VMEM((_TAB,), jnp.int32),
            pltpu.VMEM((_FTW,), jnp.float32),
            pltpu.VMEM((_C * 16,), jnp.float32),
            pltpu.VMEM((_C * _CPW,), jnp.float32),
            pltpu.SemaphoreType.DMA,
        ],
    )()
    return k(cpack, spack, fa, wsf)


def kernel(support_xyz, support_features, xyz_batch_cnt, new_xyz,
           new_xyz_grid_centers, new_xyz_batch_cnt, W, gamma, beta):
    def bf(x):
        return lax.reduce_precision(x, exponent_bits=8, mantissa_bits=7)

    gcf = new_xyz_grid_centers.reshape(_MG, 3)
    bgc = bf(gcf)

    par = jnp.repeat(new_xyz, _G, axis=0)
    sqn = (par[:, 0] * par[:, 0] + par[:, 1] * par[:, 1]) + par[:, 2] * par[:, 2]
    bpar = bf(par)

    cpack = jnp.concatenate(
        [gcf.T, bgc.T, bpar.T, sqn[None]], axis=0).reshape(-1)

    sb = support_xyz.reshape(_B, _PB, 3)
    extra = jnp.broadcast_to(support_xyz[0:3][None], (_B, 3, 3))
    padx = jnp.full((_B, _TAB - _PB - 3, 3), 1e6, jnp.float32)
    sa = jnp.concatenate([sb, extra, padx], axis=1)
    bsa = bf(sa)
    spack = jnp.concatenate(
        [sa.transpose(2, 0, 1), bsa.transpose(2, 0, 1)], axis=0).reshape(-1)

    scale = gamma * (1.0 / jnp.sqrt(jnp.float32(1.0 + 1e-5)))
    proj = _proj(support_features, W[:_C] * scale[None, :])

    fb = proj.reshape(_B, _PB, _C).transpose(0, 2, 1)
    fextra = jnp.broadcast_to(proj[0:3].T[None], (_B, _C, 3))
    fpad = jnp.zeros((_B, _C, _TAB - _PB - 3), jnp.float32)
    fa = jnp.concatenate([fb, fextra, fpad], axis=2).reshape(-1)

    wloc = jnp.concatenate(
        [(W[_C:] * scale[None, :]).T, beta[:, None],
         jnp.zeros((_C, 6), jnp.float32)], axis=1).reshape(-1)

    out = _run(cpack, spack, fa, wloc)
    return out.reshape(_NW, _C, _CPW).transpose(0, 2, 1).reshape(_MG, _C)
